# baseline (device time: 36819 ns/iter reference)
import jax
import jax.numpy as jnp
from jax import lax
from jax.experimental import pallas as pl
from jax.experimental.pallas import tpu as pltpu

N_DEV = 4
B, SQ, SKV, HQ_SHARD, DH = 2, 512, 512, 8, 64
D_MODEL = 768
RG = 256
KW = 384
WINDOW = 128
BF16 = jnp.bfloat16
F32 = jnp.float32

GROUPS = [(0, 0, 0), (0, 256, 128), (1, 0, 0), (1, 256, 128)]


def kernel(x, Wq, K_ext, V_ext, Wo):
    my = lax.axis_index("i")
    K_loc = jnp.swapaxes(
        lax.dynamic_slice_in_dim(K_ext, my * HQ_SHARD, HQ_SHARD, axis=2), 1, 2
    )
    V_loc = jnp.swapaxes(
        lax.dynamic_slice_in_dim(V_ext, my * HQ_SHARD, HQ_SHARD, axis=2), 1, 2
    )

    def body(x_ref, wq_ref, k_ref, v_ref, wo_ref, out_ref,
             ctx_ref, acc_ref, cs_ref, gath_ref, recv1_ref, recv2_ref,
             send_sems, recv_sems):
        my_pos = lax.axis_index("i")
        p1 = my_pos ^ 1
        p2 = 3 - my_pos

        s1k_e = jnp.where(
            jnp.logical_or(my_pos == 0, my_pos == 3), 0, RG // 2)
        off_e = jnp.where(my_pos <= 1, 0, RG // 4)
        s1k_o = jnp.where(my_pos <= 1, 0, RG // 2)
        off_o = jnp.where(my_pos % 2 == 0, 0, RG // 4)

        barrier_sem = pltpu.get_barrier_semaphore()
        for nbr in (p1, p2):
            pl.semaphore_signal(
                barrier_sem, inc=1,
                device_id=(nbr,), device_id_type=pl.DeviceIdType.MESH,
            )
        pl.semaphore_wait(barrier_sem, 2)

        wo_bf = wo_ref[...].astype(BF16)

        rows, descs = [], []
        for g, (b, r0, kb) in enumerate(GROUPS):
            pa, pb_ = (p1, p2) if g % 2 == 0 else (p2, p1)
            s1k = (s1k_e if g % 2 == 0 else s1k_o)
            off = (off_e if g % 2 == 0 else off_o)
            s1k_row = r0 + s1k
            s1s_row = r0 + (RG // 2 - s1k)
            qk_row = s1k_row + off
            qs_row = s1k_row + (RG // 4 - off)
            rows.append((b, s1k_row, s1s_row, qk_row, qs_row))

            def mk(src, dst, sem, tgt):
                return pltpu.make_async_remote_copy(
                    src_ref=src, dst_ref=dst,
                    send_sem=send_sems.at[g * 5 + sem],
                    recv_sem=recv_sems.at[g * 5 + sem],
                    device_id=(tgt,), device_id_type=pl.DeviceIdType.MESH,
                )
            s1 = mk(cs_ref.at[b, pl.ds(s1s_row, RG // 2)],
                    recv1_ref.at[g], 0, pa)
            s2 = mk(cs_ref.at[b, pl.ds(qs_row, RG // 4)],
                    recv2_ref.at[g], 1, pb_)
            s3 = mk(gath_ref.at[b, pl.ds(qk_row, RG // 4)],
                    gath_ref.at[b, pl.ds(qk_row, RG // 4)], 2, pb_)
            s4a = mk(gath_ref.at[b, pl.ds(qk_row, RG // 4)],
                     gath_ref.at[b, pl.ds(qk_row, RG // 4)], 3, pa)
            s4b = mk(gath_ref.at[b, pl.ds(qs_row, RG // 4)],
                     gath_ref.at[b, pl.ds(qs_row, RG // 4)], 4, pa)
            descs.append((s1, s2, s3, s4a, s4b))

        def compute(g):
            b, r0, kb = GROUPS[g]
            qi = lax.broadcasted_iota(jnp.int32, (RG, KW), 0) + r0
            ki = lax.broadcasted_iota(jnp.int32, (RG, KW), 1) + kb
            bias = jnp.where(jnp.abs(qi - ki) <= WINDOW, 0.0, -1e9
                             ).astype(F32)
            q_all = jnp.dot(
                x_ref[b, r0:r0 + RG].astype(BF16),
                wq_ref[...].astype(BF16),
                preferred_element_type=F32,
            ).astype(BF16)
            for h in range(HQ_SHARD):
                qh = q_all[:, h * DH:(h + 1) * DH]
                kh = k_ref[b, h, kb:kb + KW].astype(BF16)
                s = lax.dot_general(
                    qh, kh, (((1,), (1,)), ((), ())),
                    preferred_element_type=F32,
                )
                e = jnp.exp(s * 0.125 + bias)
                denom = jnp.sum(e, axis=1, keepdims=True)
                ctx_h = jnp.dot(
                    e.astype(BF16), v_ref[b, h, kb:kb + KW].astype(BF16),
                    preferred_element_type=F32,
                ) / denom
                ctx_ref[:, h * DH:(h + 1) * DH] = ctx_h.astype(BF16)
            pg = jnp.dot(ctx_ref[...], wo_bf, preferred_element_type=F32)
            acc_ref[b, r0:r0 + RG] = pg
            cs_ref[b, r0:r0 + RG] = pg.astype(BF16)

        def stage_a(g):
            descs[g][0].start()

        def stage_b(g):
            b, s1k_row, _, _, qs_row = rows[g]
            descs[g][0].wait()
            acc_ref[b, pl.ds(s1k_row, RG // 2)] = (
                acc_ref[b, pl.ds(s1k_row, RG // 2)]
                + recv1_ref[g].astype(F32))
            cs_ref[b, pl.ds(qs_row, RG // 4)] = (
                acc_ref[b, pl.ds(qs_row, RG // 4)].astype(BF16))
            descs[g][1].start()

        def stage_c(g):
            b, _, _, qk_row, _ = rows[g]
            descs[g][1].wait()
            gath_ref[b, pl.ds(qk_row, RG // 4)] = (
                (acc_ref[b, pl.ds(qk_row, RG // 4)]
                 + recv2_ref[g].astype(F32)).astype(BF16))
            descs[g][2].start()
            descs[g][3].start()

        def stage_d(g):
            descs[g][2].wait_recv()
            descs[g][4].start()

        def stage_e(g):
            descs[g][2].wait_send()
            descs[g][3].wait()
            descs[g][4].wait()

        compute(0); stage_a(0)
        compute(1); stage_a(1); stage_b(0)
        compute(2); stage_a(2); stage_b(1); stage_c(0)
        compute(3); stage_a(3); stage_b(2); stage_c(1); stage_d(0)
        stage_b(3); stage_c(2); stage_d(1)
        stage_c(3); stage_d(2); stage_d(3)
        for g in range(4):
            stage_e(g)

        out_ref[...] = gath_ref[...].astype(F32)

    return pl.pallas_call(
        body,
        out_shape=jax.ShapeDtypeStruct((B, SQ, D_MODEL), F32),
        in_specs=[pl.BlockSpec(memory_space=pltpu.VMEM)] * 5,
        out_specs=pl.BlockSpec(memory_space=pltpu.VMEM),
        scratch_shapes=[
            pltpu.VMEM((RG, HQ_SHARD * DH), BF16),
            pltpu.VMEM((B, SQ, D_MODEL), F32),
            pltpu.VMEM((B, SQ, D_MODEL), BF16),
            pltpu.VMEM((B, SQ, D_MODEL), BF16),
            pltpu.VMEM((4, RG // 2, D_MODEL), BF16),
            pltpu.VMEM((4, RG // 4, D_MODEL), BF16),
            pltpu.SemaphoreType.DMA((20,)),
            pltpu.SemaphoreType.DMA((20,)),
        ],
        compiler_params=pltpu.CompilerParams(collective_id=0),
    )(x, Wq, K_loc, V_loc, Wo)


# device time: 35510 ns/iter; 1.0369x vs baseline; 1.0369x over previous
import jax
import jax.numpy as jnp
from jax import lax
from jax.experimental import pallas as pl
from jax.experimental.pallas import tpu as pltpu

N_DEV = 4
B, SQ, SKV, HQ_SHARD, DH = 2, 512, 512, 8, 64
D_MODEL = 768
RG = 256
KW = 384
WINDOW = 128
BF16 = jnp.bfloat16
F32 = jnp.float32

GROUPS = [(0, 0, 0), (0, 256, 128), (1, 0, 0), (1, 256, 128)]


def kernel(x, Wq, K_ext, V_ext, Wo):
    my = lax.axis_index("i")
    K_loc = jnp.swapaxes(
        lax.dynamic_slice_in_dim(K_ext, my * HQ_SHARD, HQ_SHARD, axis=2), 1, 2
    )
    V_loc = jnp.swapaxes(
        lax.dynamic_slice_in_dim(V_ext, my * HQ_SHARD, HQ_SHARD, axis=2), 1, 2
    )

    def body(x_ref, wq_ref, k_ref, v_ref, wo_ref, out_ref,
             ctx_ref, acc_ref, cs_ref, gath_ref, recv1_ref, recv2_ref,
             recvx_ref, send_sems, recv_sems):
        my_pos = lax.axis_index("i")
        p1 = my_pos ^ 1
        p2 = 3 - my_pos
        pd = my_pos ^ 2

        s1k_e = jnp.where(
            jnp.logical_or(my_pos == 0, my_pos == 3), 0, RG // 2)
        off_e = jnp.where(my_pos <= 1, 0, RG // 4)
        s1k_o = jnp.where(my_pos <= 1, 0, RG // 2)
        off_o = jnp.where(my_pos % 2 == 0, 0, RG // 4)

        barrier_sem = pltpu.get_barrier_semaphore()
        for nbr in (p1, p2, pd):
            pl.semaphore_signal(
                barrier_sem, inc=1,
                device_id=(nbr,), device_id_type=pl.DeviceIdType.MESH,
            )
        pl.semaphore_wait(barrier_sem, 3)

        wo_bf = wo_ref[...].astype(BF16)

        rows, descs = [], []
        for g, (b, r0, kb) in list(enumerate(GROUPS))[:2]:
            pa, pb_ = (p1, p2) if g % 2 == 0 else (p2, p1)
            s1k = (s1k_e if g % 2 == 0 else s1k_o)
            off = (off_e if g % 2 == 0 else off_o)
            s1k_row = r0 + s1k
            s1s_row = r0 + (RG // 2 - s1k)
            qk_row = s1k_row + off
            qs_row = s1k_row + (RG // 4 - off)
            rows.append((b, s1k_row, s1s_row, qk_row, qs_row))

            def mk(src, dst, sem, tgt):
                return pltpu.make_async_remote_copy(
                    src_ref=src, dst_ref=dst,
                    send_sem=send_sems.at[g * 5 + sem],
                    recv_sem=recv_sems.at[g * 5 + sem],
                    device_id=(tgt,), device_id_type=pl.DeviceIdType.MESH,
                )
            s1 = mk(cs_ref.at[b, pl.ds(s1s_row, RG // 2)],
                    recv1_ref.at[g], 0, pa)
            s2 = mk(cs_ref.at[b, pl.ds(qs_row, RG // 4)],
                    recv2_ref.at[g], 1, pb_)
            s3 = mk(gath_ref.at[b, pl.ds(qk_row, RG // 4)],
                    gath_ref.at[b, pl.ds(qk_row, RG // 4)], 2, pb_)
            s4a = mk(gath_ref.at[b, pl.ds(qk_row, RG // 4)],
                     gath_ref.at[b, pl.ds(qk_row, RG // 4)], 3, pa)
            s4b = mk(gath_ref.at[b, pl.ds(qs_row, RG // 4)],
                     gath_ref.at[b, pl.ds(qs_row, RG // 4)], 4, pa)
            descs.append((s1, s2, s3, s4a, s4b))

        dd_x, dd_y, dd_rowm = [], [], []
        for gi, (b, r0, kb) in enumerate(GROUPS[2:]):
            base = 10 + gi * 6
            rowm = r0 + my_pos * (RG // 4)
            dd_rowm.append(rowm)
            xs, ys = [], []
            for slot, tgt in enumerate((p1, p2, pd)):
                xs.append(pltpu.make_async_remote_copy(
                    src_ref=cs_ref.at[b, pl.ds(r0 + tgt * (RG // 4), RG // 4)],
                    dst_ref=recvx_ref.at[gi, slot],
                    send_sem=send_sems.at[base + slot],
                    recv_sem=recv_sems.at[base + slot],
                    device_id=(tgt,), device_id_type=pl.DeviceIdType.MESH,
                ))
                ys.append(pltpu.make_async_remote_copy(
                    src_ref=gath_ref.at[b, pl.ds(rowm, RG // 4)],
                    dst_ref=gath_ref.at[b, pl.ds(rowm, RG // 4)],
                    send_sem=send_sems.at[base + 3 + slot],
                    recv_sem=recv_sems.at[base + 3 + slot],
                    device_id=(tgt,), device_id_type=pl.DeviceIdType.MESH,
                ))
            dd_x.append(xs)
            dd_y.append(ys)

        def compute(g):
            b, r0, kb = GROUPS[g]
            qi = lax.broadcasted_iota(jnp.int32, (RG, KW), 0) + r0
            ki = lax.broadcasted_iota(jnp.int32, (RG, KW), 1) + kb
            bias = jnp.where(jnp.abs(qi - ki) <= WINDOW, 0.0, -1e9
                             ).astype(F32)
            q_all = jnp.dot(
                x_ref[b, r0:r0 + RG].astype(BF16),
                wq_ref[...].astype(BF16),
                preferred_element_type=F32,
            ).astype(BF16)
            for h in range(HQ_SHARD):
                qh = q_all[:, h * DH:(h + 1) * DH]
                kh = k_ref[b, h, kb:kb + KW].astype(BF16)
                s = lax.dot_general(
                    qh, kh, (((1,), (1,)), ((), ())),
                    preferred_element_type=F32,
                )
                e = jnp.exp(s * 0.125 + bias)
                denom = jnp.sum(e, axis=1, keepdims=True)
                ctx_h = jnp.dot(
                    e.astype(BF16), v_ref[b, h, kb:kb + KW].astype(BF16),
                    preferred_element_type=F32,
                ) / denom
                ctx_ref[:, h * DH:(h + 1) * DH] = ctx_h.astype(BF16)
            pg = jnp.dot(ctx_ref[...], wo_bf, preferred_element_type=F32)
            acc_ref[b, r0:r0 + RG] = pg
            cs_ref[b, r0:r0 + RG] = pg.astype(BF16)

        def stage_a(g):
            descs[g][0].start()

        def stage_b(g):
            b, s1k_row, _, _, qs_row = rows[g]
            descs[g][0].wait()
            acc_ref[b, pl.ds(s1k_row, RG // 2)] = (
                acc_ref[b, pl.ds(s1k_row, RG // 2)]
                + recv1_ref[g].astype(F32))
            cs_ref[b, pl.ds(qs_row, RG // 4)] = (
                acc_ref[b, pl.ds(qs_row, RG // 4)].astype(BF16))
            descs[g][1].start()

        def stage_c(g):
            b, _, _, qk_row, _ = rows[g]
            descs[g][1].wait()
            gath_ref[b, pl.ds(qk_row, RG // 4)] = (
                (acc_ref[b, pl.ds(qk_row, RG // 4)]
                 + recv2_ref[g].astype(F32)).astype(BF16))
            descs[g][2].start()
            descs[g][3].start()

        def stage_d(g):
            descs[g][2].wait_recv()
            descs[g][4].start()

        def stage_e(g):
            descs[g][2].wait_send()
            descs[g][3].wait()
            descs[g][4].wait()

        def dd_scatter(gi):
            for x in dd_x[gi]:
                x.start()

        def dd_reduce(gi):
            b, r0, kb = GROUPS[2 + gi]
            for x in dd_x[gi]:
                x.wait()
            red = acc_ref[b, pl.ds(dd_rowm[gi], RG // 4)]
            for slot in range(3):
                red = red + recvx_ref[gi, slot].astype(F32)
            gath_ref[b, pl.ds(dd_rowm[gi], RG // 4)] = red.astype(BF16)
            for y in dd_y[gi]:
                y.start()

        def dd_finish(gi):
            for y in dd_y[gi]:
                y.wait()

        compute(0); stage_a(0)
        compute(1); stage_a(1); stage_b(0)
        compute(2); dd_scatter(0); stage_b(1); stage_c(0)
        compute(3); dd_scatter(1); stage_c(1); stage_d(0)
        dd_reduce(0); stage_d(1)
        dd_reduce(1)
        stage_e(0); stage_e(1)
        dd_finish(0); dd_finish(1)

        out_ref[...] = gath_ref[...].astype(F32)

    return pl.pallas_call(
        body,
        out_shape=jax.ShapeDtypeStruct((B, SQ, D_MODEL), F32),
        in_specs=[pl.BlockSpec(memory_space=pltpu.VMEM)] * 5,
        out_specs=pl.BlockSpec(memory_space=pltpu.VMEM),
        scratch_shapes=[
            pltpu.VMEM((RG, HQ_SHARD * DH), BF16),
            pltpu.VMEM((B, SQ, D_MODEL), F32),
            pltpu.VMEM((B, SQ, D_MODEL), BF16),
            pltpu.VMEM((B, SQ, D_MODEL), BF16),
            pltpu.VMEM((2, RG // 2, D_MODEL), BF16),
            pltpu.VMEM((2, RG // 4, D_MODEL), BF16),
            pltpu.VMEM((2, 3, RG // 4, D_MODEL), BF16),
            pltpu.SemaphoreType.DMA((22,)),
            pltpu.SemaphoreType.DMA((22,)),
        ],
        compiler_params=pltpu.CompilerParams(collective_id=0),
    )(x, Wq, K_loc, V_loc, Wo)


# device time: 33969 ns/iter; 1.0839x vs baseline; 1.0454x over previous
import jax
import jax.numpy as jnp
from jax import lax
from jax.experimental import pallas as pl
from jax.experimental.pallas import tpu as pltpu

N_DEV = 4
B, SQ, SKV, HQ_SHARD, DH = 2, 512, 512, 8, 64
D_MODEL = 768
RG = 256
KW = 384
WINDOW = 128
BF16 = jnp.bfloat16
F32 = jnp.float32

GROUPS = [(0, 0, 0), (0, 256, 128), (1, 0, 0), (1, 256, 128)]


def kernel(x, Wq, K_ext, V_ext, Wo):
    my = lax.axis_index("i")
    K_loc = lax.dynamic_slice_in_dim(
        K_ext.reshape(B, SKV, 32 * DH), my * (HQ_SHARD * DH),
        HQ_SHARD * DH, axis=2).astype(BF16)
    V_loc = lax.dynamic_slice_in_dim(
        V_ext.reshape(B, SKV, 32 * DH), my * (HQ_SHARD * DH),
        HQ_SHARD * DH, axis=2).astype(BF16)

    def body(x_ref, wq_ref, k_ref, v_ref, wo_ref, out_ref,
             ctx_ref, acc_ref, cs_ref, gath_ref, recv1_ref, recv2_ref,
             recvx_ref, send_sems, recv_sems):
        my_pos = lax.axis_index("i")
        p1 = my_pos ^ 1
        p2 = 3 - my_pos
        pd = my_pos ^ 2

        s1k_e = jnp.where(
            jnp.logical_or(my_pos == 0, my_pos == 3), 0, RG // 2)
        off_e = jnp.where(my_pos <= 1, 0, RG // 4)
        s1k_o = jnp.where(my_pos <= 1, 0, RG // 2)
        off_o = jnp.where(my_pos % 2 == 0, 0, RG // 4)

        barrier_sem = pltpu.get_barrier_semaphore()
        for nbr in (p1, p2, pd):
            pl.semaphore_signal(
                barrier_sem, inc=1,
                device_id=(nbr,), device_id_type=pl.DeviceIdType.MESH,
            )
        pl.semaphore_wait(barrier_sem, 3)

        wo_bf = wo_ref[...].astype(BF16)

        rows, descs = [], []
        for g, (b, r0, kb) in list(enumerate(GROUPS))[:2]:
            pa, pb_ = (p1, p2) if g % 2 == 0 else (p2, p1)
            s1k = (s1k_e if g % 2 == 0 else s1k_o)
            off = (off_e if g % 2 == 0 else off_o)
            s1k_row = r0 + s1k
            s1s_row = r0 + (RG // 2 - s1k)
            qk_row = s1k_row + off
            qs_row = s1k_row + (RG // 4 - off)
            rows.append((b, s1k_row, s1s_row, qk_row, qs_row))

            def mk(src, dst, sem, tgt):
                return pltpu.make_async_remote_copy(
                    src_ref=src, dst_ref=dst,
                    send_sem=send_sems.at[g * 5 + sem],
                    recv_sem=recv_sems.at[g * 5 + sem],
                    device_id=(tgt,), device_id_type=pl.DeviceIdType.MESH,
                )
            s1 = mk(cs_ref.at[b, pl.ds(s1s_row, RG // 2)],
                    recv1_ref.at[g], 0, pa)
            s2 = mk(cs_ref.at[b, pl.ds(qs_row, RG // 4)],
                    recv2_ref.at[g], 1, pb_)
            s3 = mk(gath_ref.at[b, pl.ds(qk_row, RG // 4)],
                    gath_ref.at[b, pl.ds(qk_row, RG // 4)], 2, pb_)
            s4a = mk(gath_ref.at[b, pl.ds(qk_row, RG // 4)],
                     gath_ref.at[b, pl.ds(qk_row, RG // 4)], 3, pa)
            s4b = mk(gath_ref.at[b, pl.ds(qs_row, RG // 4)],
                     gath_ref.at[b, pl.ds(qs_row, RG // 4)], 4, pa)
            descs.append((s1, s2, s3, s4a, s4b))

        dd_x, dd_y, dd_rowm = [], [], []
        for gi, (b, r0, kb) in enumerate(GROUPS[2:]):
            base = 10 + gi * 6
            rowm = r0 + my_pos * (RG // 4)
            dd_rowm.append(rowm)
            xs, ys = [], []
            for slot, tgt in enumerate((p1, p2, pd)):
                xs.append(pltpu.make_async_remote_copy(
                    src_ref=cs_ref.at[b, pl.ds(r0 + tgt * (RG // 4), RG // 4)],
                    dst_ref=recvx_ref.at[gi, slot],
                    send_sem=send_sems.at[base + slot],
                    recv_sem=recv_sems.at[base + slot],
                    device_id=(tgt,), device_id_type=pl.DeviceIdType.MESH,
                ))
                ys.append(pltpu.make_async_remote_copy(
                    src_ref=gath_ref.at[b, pl.ds(rowm, RG // 4)],
                    dst_ref=gath_ref.at[b, pl.ds(rowm, RG // 4)],
                    send_sem=send_sems.at[base + 3 + slot],
                    recv_sem=recv_sems.at[base + 3 + slot],
                    device_id=(tgt,), device_id_type=pl.DeviceIdType.MESH,
                ))
            dd_x.append(xs)
            dd_y.append(ys)

        def compute(g, xs=None):
            b, r0, kb = GROUPS[g]
            qi = lax.broadcasted_iota(jnp.int32, (RG, KW), 0) + r0
            ki = lax.broadcasted_iota(jnp.int32, (RG, KW), 1) + kb
            bias = jnp.where(jnp.abs(qi - ki) <= WINDOW, 0.0, -1e9
                             ).astype(F32)
            q_all = (jnp.dot(
                x_ref[b, r0:r0 + RG].astype(BF16),
                wq_ref[...].astype(BF16),
                preferred_element_type=F32,
            ) * (0.125 * 1.4426950408889634)).astype(BF16)
            for h in range(HQ_SHARD):
                qh = q_all[:, h * DH:(h + 1) * DH]
                kh = k_ref[b, kb:kb + KW, h * DH:(h + 1) * DH]
                s = lax.dot_general(
                    qh, kh, (((1,), (1,)), ((), ())),
                    preferred_element_type=F32,
                )
                e = jnp.exp2(s + bias)
                denom = jnp.sum(e, axis=1, keepdims=True)
                ctx_h = jnp.dot(
                    e.astype(BF16), v_ref[b, kb:kb + KW, h * DH:(h + 1) * DH],
                    preferred_element_type=F32,
                ) / denom
                ctx_ref[:, h * DH:(h + 1) * DH] = ctx_h.astype(BF16)
            if xs is None:
                pg = jnp.dot(ctx_ref[...], wo_bf, preferred_element_type=F32)
                acc_ref[b, r0:r0 + RG] = pg
                cs_ref[b, r0:r0 + RG] = pg.astype(BF16)
            else:
                for tgt, xdesc in xs:
                    loc = tgt * (RG // 4)
                    pg = jnp.dot(ctx_ref[pl.ds(loc, RG // 4)], wo_bf,
                                 preferred_element_type=F32)
                    acc_ref[b, pl.ds(r0 + loc, RG // 4)] = pg
                    cs_ref[b, pl.ds(r0 + loc, RG // 4)] = pg.astype(BF16)
                    if xdesc is not None:
                        xdesc.start()

        def stage_a(g):
            descs[g][0].start()

        def stage_b(g):
            b, s1k_row, _, _, qs_row = rows[g]
            descs[g][0].wait()
            acc_ref[b, pl.ds(s1k_row, RG // 2)] = (
                acc_ref[b, pl.ds(s1k_row, RG // 2)]
                + recv1_ref[g].astype(F32))
            cs_ref[b, pl.ds(qs_row, RG // 4)] = (
                acc_ref[b, pl.ds(qs_row, RG // 4)].astype(BF16))
            descs[g][1].start()

        def stage_c(g):
            b, _, _, qk_row, _ = rows[g]
            descs[g][1].wait()
            gath_ref[b, pl.ds(qk_row, RG // 4)] = (
                (acc_ref[b, pl.ds(qk_row, RG // 4)]
                 + recv2_ref[g].astype(F32)).astype(BF16))
            descs[g][2].start()
            descs[g][3].start()

        def stage_d(g):
            descs[g][2].wait_recv()
            descs[g][4].start()

        def stage_e(g):
            descs[g][2].wait_send()
            descs[g][3].wait()
            descs[g][4].wait()

        def dd_scatter(gi):
            for x in dd_x[gi]:
                x.start()

        def dd_reduce(gi):
            b, r0, kb = GROUPS[2 + gi]
            for x in dd_x[gi]:
                x.wait()
            red = acc_ref[b, pl.ds(dd_rowm[gi], RG // 4)]
            for slot in range(3):
                red = red + recvx_ref[gi, slot].astype(F32)
            gath_ref[b, pl.ds(dd_rowm[gi], RG // 4)] = red.astype(BF16)
            for y in dd_y[gi]:
                y.start()

        def dd_finish(gi):
            for y in dd_y[gi]:
                y.wait()

        xs2 = list(zip((p1, p2, pd), dd_x[0])) + [(my_pos, None)]
        xs3 = list(zip((p1, p2, pd), dd_x[1])) + [(my_pos, None)]
        compute(0); stage_a(0)
        compute(1); stage_a(1); stage_b(0)
        compute(2, xs=xs2); stage_b(1); stage_c(0)
        compute(3, xs=xs3); stage_c(1); stage_d(0)
        dd_reduce(0)
        dd_reduce(1)
        stage_d(1)
        stage_e(0); stage_e(1)
        dd_finish(0); dd_finish(1)

        out_ref[...] = gath_ref[...].astype(F32)

    return pl.pallas_call(
        body,
        out_shape=jax.ShapeDtypeStruct((B, SQ, D_MODEL), F32),
        in_specs=[pl.BlockSpec(memory_space=pltpu.VMEM)] * 5,
        out_specs=pl.BlockSpec(memory_space=pltpu.VMEM),
        scratch_shapes=[
            pltpu.VMEM((RG, HQ_SHARD * DH), BF16),
            pltpu.VMEM((B, SQ, D_MODEL), F32),
            pltpu.VMEM((B, SQ, D_MODEL), BF16),
            pltpu.VMEM((B, SQ, D_MODEL), BF16),
            pltpu.VMEM((2, RG // 2, D_MODEL), BF16),
            pltpu.VMEM((2, RG // 4, D_MODEL), BF16),
            pltpu.VMEM((2, 3, RG // 4, D_MODEL), BF16),
            pltpu.SemaphoreType.DMA((22,)),
            pltpu.SemaphoreType.DMA((22,)),
        ],
        compiler_params=pltpu.CompilerParams(collective_id=0),
    )(x, Wq, K_loc, V_loc, Wo)
